# baseline (speedup 1.0000x reference)
import jax
import jax.numpy as jnp
from jax import lax
from jax.experimental import pallas as pl
from jax.experimental.pallas import tpu as pltpu

_B, _C, _H, _W = 64, 256, 64, 32
_RH, _RW = 3, 3
_NB = 2


def _sigmoid(v):
    return 1.0 / (1.0 + jnp.exp(-v))


def _dgt(a, b):
    return lax.dot_general(a, b, (((1,), (1,)), ((), ())),
                           preferred_element_type=jnp.float32)


def _head(g, conv_w, conv_b, fc1_w, fc1_b, fc2g_ref, fc2b, limit):
    s = _dgt(g, conv_w) + conv_b[None, :]
    th = jax.nn.relu(_dgt(s, fc1_w) + fc1_b[None, :])
    a1 = 2.0 * _sigmoid(_dgt(th, fc2g_ref[0]) + fc2b[0][None, :])
    a2 = 2.0 * _sigmoid(_dgt(th, fc2g_ref[1]) + fc2b[1][None, :]) - 1.0
    b1 = _sigmoid(_dgt(th, fc2g_ref[2]) + fc2b[2][None, :]) - 0.5
    b2 = _sigmoid(_dgt(th, fc2g_ref[3]) + fc2b[3][None, :]) - 0.5
    dy = jnp.maximum(s * a1 + b1, s * a2 + b2)
    coord = jnp.minimum(jnp.ceil(_H * _sigmoid(dy)), float(limit))
    return coord.astype(jnp.int32)


def _body(x_ref, w1_ref, w2_ref, convh_w_ref, convh_b_ref, fc1h_w_ref, fc1h_b_ref,
          fc2gh_ref, fc2bh_ref, convw_w_ref, convw_b_ref, fc1w_w_ref, fc1w_b_ref,
          fc2gw_ref, fc2bw_ref, out_ref):
    x = x_ref[...]
    m = jnp.sum(jnp.sum(x, axis=1), axis=1) * (1.0 / (_H * _W))
    t = jax.nn.relu(_dgt(m, w1_ref[...]))
    y = _sigmoid(_dgt(t, w2_ref[...]))
    g = m * y
    sx = _head(g, convh_w_ref[...], convh_b_ref[...], fc1h_w_ref[...],
               fc1h_b_ref[...], fc2gh_ref, fc2bh_ref[...], _H - _RH)
    sy = _head(g, convw_w_ref[...], convw_b_ref[...], fc1w_w_ref[...],
               fc1w_b_ref[...], fc2gw_ref, fc2bw_ref[...], _W - _RW)

    ii = lax.broadcasted_iota(jnp.int32, (_NB, _H, 1, _C), 1)
    jj = lax.broadcasted_iota(jnp.int32, (_NB, 1, _W, _C), 2)
    row_bad = (ii - sx[:, None, None, :]).astype(jnp.uint32) < jnp.uint32(_RH)
    col_bad = (jj - sy[:, None, None, :]).astype(jnp.uint32) < jnp.uint32(_RW)
    out_ref[...] = jnp.where(row_bad & col_bad, 0.0, x * y[:, None, None, :])


def kernel(x, se_w1, se_w2, convh_w, convh_b, dyh_fc1_w, dyh_fc1_b, dyh_fc2_w, dyh_fc2_b,
           convw_w, convw_b, dyw_fc1_w, dyw_fc1_b, dyw_fc2_w, dyw_fc2_b):
    b, c, h, w = x.shape
    xt = jnp.transpose(x, (0, 2, 3, 1))
    red = dyh_fc1_w.shape[0]
    gm = lambda wf: wf.reshape(c, 4, red).transpose(1, 0, 2)
    gb = lambda bf: bf.reshape(c, 4).T
    wgts = (se_w1, se_w2,
            convh_w, convh_b, dyh_fc1_w, dyh_fc1_b, gm(dyh_fc2_w), gb(dyh_fc2_b),
            convw_w, convw_b, dyw_fc1_w, dyw_fc1_b, gm(dyw_fc2_w), gb(dyw_fc2_b))
    full = lambda a: pl.BlockSpec(a.shape, lambda i: (0,) * a.ndim)
    outt = pl.pallas_call(
        _body,
        grid=(b // _NB,),
        in_specs=[pl.BlockSpec((_NB, h, w, c), lambda i: (i, 0, 0, 0))]
        + [full(a) for a in wgts],
        out_specs=pl.BlockSpec((_NB, h, w, c), lambda i: (i, 0, 0, 0)),
        out_shape=jax.ShapeDtypeStruct((b, h, w, c), jnp.float32),
        compiler_params=pltpu.CompilerParams(
            dimension_semantics=("arbitrary",),
        ),
    )(xt, *wgts)
    return jnp.transpose(outt, (0, 3, 1, 2))


# speedup vs baseline: 1.2089x; 1.2089x over previous
import jax
import jax.numpy as jnp
from jax import lax
from jax.experimental import pallas as pl
from jax.experimental.pallas import tpu as pltpu

_B, _C, _H, _W = 64, 256, 64, 32
_RH, _RW = 3, 3
_NB = 4


def _sigmoid(v):
    return 1.0 / (1.0 + jnp.exp(-v))


def _dgt(a, b):
    return lax.dot_general(a, b, (((1,), (1,)), ((), ())),
                           preferred_element_type=jnp.float32)


def _head(g, conv_w, conv_b, fc1_w, fc1_b, fc2g_ref, fc2b, limit):
    s = _dgt(g, conv_w) + conv_b[None, :]
    th = jax.nn.relu(_dgt(s, fc1_w) + fc1_b[None, :])
    a1 = 2.0 * _sigmoid(_dgt(th, fc2g_ref[0]) + fc2b[0][None, :])
    a2 = 2.0 * _sigmoid(_dgt(th, fc2g_ref[1]) + fc2b[1][None, :]) - 1.0
    b1 = _sigmoid(_dgt(th, fc2g_ref[2]) + fc2b[2][None, :]) - 0.5
    b2 = _sigmoid(_dgt(th, fc2g_ref[3]) + fc2b[3][None, :]) - 0.5
    dy = jnp.maximum(s * a1 + b1, s * a2 + b2)
    coord = jnp.minimum(jnp.ceil(_H * _sigmoid(dy)), float(limit))
    return coord.astype(jnp.int32)


def _body(x_ref, w1_ref, w2_ref, convh_w_ref, convh_b_ref, fc1h_w_ref, fc1h_b_ref,
          fc2gh_ref, fc2bh_ref, convw_w_ref, convw_b_ref, fc1w_w_ref, fc1w_b_ref,
          fc2gw_ref, fc2bw_ref, out_ref):
    x = x_ref[...]
    m = jnp.sum(jnp.sum(x, axis=1), axis=1) * (1.0 / (_H * _W))
    t = jax.nn.relu(_dgt(m, w1_ref[...]))
    y = _sigmoid(_dgt(t, w2_ref[...]))
    g = m * y
    sx = _head(g, convh_w_ref[...], convh_b_ref[...], fc1h_w_ref[...],
               fc1h_b_ref[...], fc2gh_ref, fc2bh_ref[...], _H - _RH)
    sy = _head(g, convw_w_ref[...], convw_b_ref[...], fc1w_w_ref[...],
               fc1w_b_ref[...], fc2gw_ref, fc2bw_ref[...], _W - _RW)

    ii = lax.broadcasted_iota(jnp.int32, (_NB, _H, _W, _C), 1)
    jj = lax.broadcasted_iota(jnp.int32, (_NB, _H, _W, _C), 2)
    row_bad = (ii - sx[:, None, None, :]).astype(jnp.uint32) < jnp.uint32(_RH)
    col_bad = (jj - sy[:, None, None, :]).astype(jnp.uint32) < jnp.uint32(_RW)
    out_ref[...] = jnp.where(row_bad & col_bad, 0.0, x * y[:, None, None, :])


def kernel(x, se_w1, se_w2, convh_w, convh_b, dyh_fc1_w, dyh_fc1_b, dyh_fc2_w, dyh_fc2_b,
           convw_w, convw_b, dyw_fc1_w, dyw_fc1_b, dyw_fc2_w, dyw_fc2_b):
    b, c, h, w = x.shape
    xt = jnp.transpose(x, (0, 2, 3, 1))
    red = dyh_fc1_w.shape[0]
    gm = lambda wf: wf.reshape(c, 4, red).transpose(1, 0, 2)
    gb = lambda bf: bf.reshape(c, 4).T
    wgts = (se_w1, se_w2,
            convh_w, convh_b, dyh_fc1_w, dyh_fc1_b, gm(dyh_fc2_w), gb(dyh_fc2_b),
            convw_w, convw_b, dyw_fc1_w, dyw_fc1_b, gm(dyw_fc2_w), gb(dyw_fc2_b))
    full = lambda a: pl.BlockSpec(a.shape, lambda i: (0,) * a.ndim)
    outt = pl.pallas_call(
        _body,
        grid=(b // _NB,),
        in_specs=[pl.BlockSpec((_NB, h, w, c), lambda i: (i, 0, 0, 0))]
        + [full(a) for a in wgts],
        out_specs=pl.BlockSpec((_NB, h, w, c), lambda i: (i, 0, 0, 0)),
        out_shape=jax.ShapeDtypeStruct((b, h, w, c), jnp.float32),
        compiler_params=pltpu.CompilerParams(
            dimension_semantics=("arbitrary",),
            vmem_limit_bytes=64 * 1024 * 1024,
        ),
    )(xt, *wgts)
    return jnp.transpose(outt, (0, 3, 1, 2))
